# baseline (speedup 1.0000x reference)
import functools

import jax
import jax.numpy as jnp
from jax import lax
from jax.experimental import pallas as pl
from jax.experimental.pallas import tpu as pltpu
from jax.experimental.pallas import tpu_sc as plsc

NN = 10000
EE = 320000
DD = 128
HH = 8
FF = 8
OUTD = 128

NC = 2
NS = 16
NW = NC * NS
EPT = EE // NW
CH = 40
NCHUNK = EPT // CH
NP = 10240
RPT = NP // NS

RB = 400
GRID = NN // RB


def _tc_prep_body(x_ref, w0_ref, al_ref, ar_ref, fel_ref, er_ref):
    feat = jnp.dot(x_ref[...], w0_ref[...], preferred_element_type=jnp.float32)
    el64 = jnp.dot(feat, al_ref[...], preferred_element_type=jnp.float32)
    er64 = jnp.dot(feat, ar_ref[...], preferred_element_type=jnp.float32)
    fel_ref[...] = jnp.concatenate([feat, el64], axis=1)
    er_ref[...] = jnp.concatenate([er64, jnp.zeros_like(er64)], axis=1)


def _tc_mid_body(acc_ref, w1bd_ref, a1l_ref, a1r_ref, wlin_ref, tau_ref,
                 fel1_ref, er1_ref, z8_ref):
    s = acc_ref[0] + acc_ref[1]
    h = s[:, :64] / (s[:, 64:] + 1e-9)
    featg = jnp.dot(h, w1bd_ref[...], preferred_element_type=jnp.float32)
    el1 = jnp.dot(featg, a1l_ref[...], preferred_element_type=jnp.float32)
    er1 = jnp.dot(featg, a1r_ref[...], preferred_element_type=jnp.float32)
    lw = jnp.dot(h, wlin_ref[...], preferred_element_type=jnp.float32)
    m = jnp.max(lw, axis=1, keepdims=True)
    ssum = jnp.sum(jnp.exp(lw - m), axis=1, keepdims=True)
    z = tau_ref[0, 0] + tau_ref[0, 1] / ssum
    fel1_ref[...] = jnp.concatenate([featg, el1], axis=1)
    er1_ref[...] = jnp.concatenate([er1, jnp.zeros_like(er1)], axis=1)
    z8_ref[...] = jnp.broadcast_to(z, (z.shape[0], 8))


def _tc_out_body(acc_ref, fel1_ref, z8_ref, wlin_ref, out_ref):
    s = acc_ref[0] + acc_ref[1]
    agg = s[:, :64] / (s[:, 64:] + 1e-9)
    featg = fel1_ref[:, :64]
    z = z8_ref[:, :1]
    h2 = z * agg + (1.0 - z) * featg
    out_ref[...] = jnp.dot(h2, wlin_ref[...], preferred_element_type=jnp.float32)


def _full(shape):
    return pl.BlockSpec(shape, lambda i: (0,) * len(shape))


_tc_prep = pl.pallas_call(
    _tc_prep_body,
    grid=(GRID,),
    in_specs=[
        pl.BlockSpec((RB, DD), lambda i: (i, 0)),
        _full((DD, 64)), _full((64, 64)), _full((64, 64)),
    ],
    out_specs=[
        pl.BlockSpec((RB, 128), lambda i: (i, 0)),
        pl.BlockSpec((RB, 128), lambda i: (i, 0)),
    ],
    out_shape=[
        jax.ShapeDtypeStruct((NN, 128), jnp.float32),
        jax.ShapeDtypeStruct((NN, 128), jnp.float32),
    ],
)

_tc_mid = pl.pallas_call(
    _tc_mid_body,
    grid=(GRID,),
    in_specs=[
        pl.BlockSpec((2, RB, 128), lambda i: (0, i, 0)),
        _full((64, 64)), _full((64, 64)), _full((64, 64)), _full((64, OUTD)),
        _full((1, 2)),
    ],
    out_specs=[
        pl.BlockSpec((RB, 128), lambda i: (i, 0)),
        pl.BlockSpec((RB, 128), lambda i: (i, 0)),
        pl.BlockSpec((RB, 8), lambda i: (i, 0)),
    ],
    out_shape=[
        jax.ShapeDtypeStruct((NN, 128), jnp.float32),
        jax.ShapeDtypeStruct((NN, 128), jnp.float32),
        jax.ShapeDtypeStruct((NN, 8), jnp.float32),
    ],
)

_tc_out = pl.pallas_call(
    _tc_out_body,
    grid=(GRID,),
    in_specs=[
        pl.BlockSpec((2, RB, 128), lambda i: (0, i, 0)),
        pl.BlockSpec((RB, 128), lambda i: (i, 0)),
        pl.BlockSpec((RB, 8), lambda i: (i, 0)),
        _full((64, OUTD)),
    ],
    out_specs=pl.BlockSpec((RB, OUTD), lambda i: (i, 0)),
    out_shape=jax.ShapeDtypeStruct((NN, OUTD), jnp.float32),
)


def _sc_edge_body(fel_hbm, er_hbm, src_hbm, dst_hbm, zrow_hbm, acc_out,
                  acc_sh, src1d, dst1d, fel0, er0, fel1, er1, didx0, didx1,
                  sem0, sem1, semi0, semi1):
    c = lax.axis_index("c")
    s = lax.axis_index("s")
    wid = c * NS + s
    base = wid * EPT
    pltpu.sync_copy(zrow_hbm, acc_sh.at[pl.ds(s * RPT, RPT)])
    pltpu.sync_copy(src_hbm.at[pl.ds(base, EPT)], src1d)
    pltpu.sync_copy(dst_hbm.at[pl.ds(base, EPT)], dst1d)
    plsc.subcore_barrier()

    def gather(i, fel_b, er_b, sem):
        pltpu.async_copy(fel_hbm.at[src1d.at[pl.ds(i * CH, CH)]], fel_b, sem)
        pltpu.async_copy(er_hbm.at[dst1d.at[pl.ds(i * CH, CH)]], er_b, sem)

    def gwait(i, fel_b, er_b, sem):
        pltpu.make_async_copy(
            fel_hbm.at[src1d.at[pl.ds(i * CH, CH)]], fel_b, sem).wait()
        pltpu.make_async_copy(
            er_hbm.at[dst1d.at[pl.ds(i * CH, CH)]], er_b, sem).wait()

    def istart(i, didx_b, semi):
        pltpu.async_copy(dst_hbm.at[pl.ds(base + i * CH, CH)], didx_b, semi)

    def iwait(i, didx_b, semi):
        pltpu.make_async_copy(
            dst_hbm.at[pl.ds(base + i * CH, CH)], didx_b, semi).wait()

    def compute_scatter(i, fel_t, er_t, didx_b, semi):
        @plsc.parallel_loop(0, CH, unroll=8)
        def _edge_loop(e):
            for k in range(4):
                f = fel_t[e, pl.ds(k * 16, 16)]
                a = fel_t[e, pl.ds(64 + k * 16, 16)]
                b = er_t[e, pl.ds(k * 16, 16)]
                t = a + b
                t = jnp.maximum(t, 0.2 * t)
                ex = jnp.exp(t)
                fel_t[e, pl.ds(64 + k * 16, 16)] = ex
                fel_t[e, pl.ds(k * 16, 16)] = ex * f
        iwait(i, didx_b, semi)
        pltpu.sync_copy(fel_t, acc_sh.at[didx_b], add=True)

    istart(0, didx0, semi0)
    istart(1, didx1, semi1)
    gather(0, fel0, er0, sem0)

    def pair(j, carry):
        i0 = 2 * j
        gather(i0 + 1, fel1, er1, sem1)
        gwait(i0, fel0, er0, sem0)
        compute_scatter(i0, fel0, er0, didx0, semi0)
        istart(i0 + 2, didx0, semi0)
        gather(i0 + 2, fel0, er0, sem0)
        gwait(i0 + 1, fel1, er1, sem1)
        compute_scatter(i0 + 1, fel1, er1, didx1, semi1)
        istart(i0 + 3, didx1, semi1)
        return carry

    lax.fori_loop(0, NCHUNK // 2 - 1, pair, 0)
    i0 = NCHUNK - 2
    gather(i0 + 1, fel1, er1, sem1)
    gwait(i0, fel0, er0, sem0)
    compute_scatter(i0, fel0, er0, didx0, semi0)
    gwait(i0 + 1, fel1, er1, sem1)
    compute_scatter(i0 + 1, fel1, er1, didx1, semi1)

    plsc.subcore_barrier()
    pltpu.sync_copy(acc_sh.at[pl.ds(s * RPT, RPT)],
                    acc_out.at[c, pl.ds(s * RPT, RPT)])


@functools.cache
def _sc_edge_kernel():
  return pl.kernel(
    _sc_edge_body,
    out_type=jax.ShapeDtypeStruct((NC, NP, 128), jnp.float32),
    mesh=plsc.VectorSubcoreMesh(core_axis_name="c", subcore_axis_name="s",
                                num_cores=NC, num_subcores=NS),
    scratch_types=[
        pltpu.VMEM_SHARED((NP, 128), jnp.float32),
        pltpu.VMEM((EPT,), jnp.int32),
        pltpu.VMEM((EPT,), jnp.int32),
        pltpu.VMEM((CH, 128), jnp.float32),
        pltpu.VMEM((CH, 128), jnp.float32),
        pltpu.VMEM((CH, 128), jnp.float32),
        pltpu.VMEM((CH, 128), jnp.float32),
        pltpu.VMEM((CH,), jnp.int32),
        pltpu.VMEM((CH,), jnp.int32),
        pltpu.SemaphoreType.DMA,
        pltpu.SemaphoreType.DMA,
        pltpu.SemaphoreType.DMA,
        pltpu.SemaphoreType.DMA,
    ],
  )


def kernel(x, edge_index, init_weight_y, W0, al0, ar0, W1, attn_l, attn_r,
           tau1, tau2, W_lin):
    src = edge_index[0]
    dst = edge_index[1]

    bcast = jnp.kron(jnp.eye(HH, dtype=jnp.float32),
                     jnp.ones((FF, FF), dtype=jnp.float32))
    Al = al0.reshape(HH * FF, 1) * bcast
    Ar = ar0.reshape(HH * FF, 1) * bcast
    A1l = attn_l.reshape(HH * FF, 1) * bcast
    A1r = attn_r.reshape(HH * FF, 1) * bcast
    W1bd = jnp.kron(jnp.eye(HH, dtype=jnp.float32), W1)
    taus = jnp.stack([tau1, tau2]).reshape(1, 2).astype(jnp.float32)
    zrow = jnp.zeros((RPT, 128), jnp.float32)

    sc_edge = _sc_edge_kernel()
    fel, er64 = _tc_prep(x, W0, Al, Ar)
    acc0 = sc_edge(fel, er64, src, dst, zrow)
    fel1, er1, z8 = _tc_mid(acc0, W1bd, A1l, A1r, W_lin, taus)
    acc1 = sc_edge(fel1, er1, src, dst, zrow)
    out = _tc_out(acc1, fel1, z8, W_lin)
    return out


# speedup vs baseline: 1.1464x; 1.1464x over previous
import functools

import jax
import jax.numpy as jnp
from jax import lax
from jax.experimental import pallas as pl
from jax.experimental.pallas import tpu as pltpu
from jax.experimental.pallas import tpu_sc as plsc

NN = 10000
EE = 320000
DD = 128
HH = 8
FF = 8
OUTD = 128

NC = 2
NS = 16
NW = NC * NS
EPT = EE // NW
CH = 80
NCHUNK = EPT // CH
NP = 10112
RPT = NP // NS

RB = 400
GRID = NN // RB


def _tc_prep_body(x_ref, w0_ref, al_ref, ar_ref, fel_ref, er_ref):
    feat = jnp.dot(x_ref[...], w0_ref[...], preferred_element_type=jnp.float32)
    el64 = jnp.dot(feat, al_ref[...], preferred_element_type=jnp.float32)
    er64 = jnp.dot(feat, ar_ref[...], preferred_element_type=jnp.float32)
    fel_ref[...] = jnp.concatenate([feat, el64], axis=1)
    er_ref[...] = jnp.concatenate([er64, jnp.zeros_like(er64)], axis=1)


def _tc_mid_body(acc_ref, w1bd_ref, a1l_ref, a1r_ref, wlin_ref, tau_ref,
                 fel1_ref, er1_ref, z8_ref):
    s = acc_ref[0] + acc_ref[1]
    h = s[:, :64] / (s[:, 64:] + 1e-9)
    featg = jnp.dot(h, w1bd_ref[...], preferred_element_type=jnp.float32)
    el1 = jnp.dot(featg, a1l_ref[...], preferred_element_type=jnp.float32)
    er1 = jnp.dot(featg, a1r_ref[...], preferred_element_type=jnp.float32)
    lw = jnp.dot(h, wlin_ref[...], preferred_element_type=jnp.float32)
    m = jnp.max(lw, axis=1, keepdims=True)
    ssum = jnp.sum(jnp.exp(lw - m), axis=1, keepdims=True)
    z = tau_ref[0, 0] + tau_ref[0, 1] / ssum
    fel1_ref[...] = jnp.concatenate([featg, el1], axis=1)
    er1_ref[...] = jnp.concatenate([er1, jnp.zeros_like(er1)], axis=1)
    z8_ref[...] = jnp.broadcast_to(z, (z.shape[0], 8))


def _tc_out_body(acc_ref, fel1_ref, z8_ref, wlin_ref, out_ref):
    s = acc_ref[0] + acc_ref[1]
    agg = s[:, :64] / (s[:, 64:] + 1e-9)
    featg = fel1_ref[:, :64]
    z = z8_ref[:, :1]
    h2 = z * agg + (1.0 - z) * featg
    out_ref[...] = jnp.dot(h2, wlin_ref[...], preferred_element_type=jnp.float32)


def _full(shape):
    return pl.BlockSpec(shape, lambda i: (0,) * len(shape))


_tc_prep = pl.pallas_call(
    _tc_prep_body,
    grid=(GRID,),
    in_specs=[
        pl.BlockSpec((RB, DD), lambda i: (i, 0)),
        _full((DD, 64)), _full((64, 64)), _full((64, 64)),
    ],
    out_specs=[
        pl.BlockSpec((RB, 128), lambda i: (i, 0)),
        pl.BlockSpec((RB, 128), lambda i: (i, 0)),
    ],
    out_shape=[
        jax.ShapeDtypeStruct((NN, 128), jnp.float32),
        jax.ShapeDtypeStruct((NN, 128), jnp.float32),
    ],
)

_tc_mid = pl.pallas_call(
    _tc_mid_body,
    grid=(GRID,),
    in_specs=[
        pl.BlockSpec((2, RB, 128), lambda i: (0, i, 0)),
        _full((64, 64)), _full((64, 64)), _full((64, 64)), _full((64, OUTD)),
        _full((1, 2)),
    ],
    out_specs=[
        pl.BlockSpec((RB, 128), lambda i: (i, 0)),
        pl.BlockSpec((RB, 128), lambda i: (i, 0)),
        pl.BlockSpec((RB, 8), lambda i: (i, 0)),
    ],
    out_shape=[
        jax.ShapeDtypeStruct((NN, 128), jnp.float32),
        jax.ShapeDtypeStruct((NN, 128), jnp.float32),
        jax.ShapeDtypeStruct((NN, 8), jnp.float32),
    ],
)

_tc_out = pl.pallas_call(
    _tc_out_body,
    grid=(GRID,),
    in_specs=[
        pl.BlockSpec((2, RB, 128), lambda i: (0, i, 0)),
        pl.BlockSpec((RB, 128), lambda i: (i, 0)),
        pl.BlockSpec((RB, 8), lambda i: (i, 0)),
        _full((64, OUTD)),
    ],
    out_specs=pl.BlockSpec((RB, OUTD), lambda i: (i, 0)),
    out_shape=jax.ShapeDtypeStruct((NN, OUTD), jnp.float32),
)


def _sc_edge_body(fel_hbm, er_hbm, src_hbm, dst_hbm, zrow_hbm, acc_out,
                  acc_sh, fel0, er0, fel1, er1,
                  sidx0, sidx1, sidx2, sidx3, didx0, didx1, didx2, didx3,
                  sem0, sem1, semi0, semi1, semi2, semi3):
    c = lax.axis_index("c")
    s = lax.axis_index("s")
    wid = c * NS + s
    base = wid * EPT
    pltpu.sync_copy(zrow_hbm, acc_sh.at[pl.ds(s * RPT, RPT)])
    plsc.subcore_barrier()

    rows = [(fel0, er0, sem0), (fel1, er1, sem1)]
    slots = [(sidx0, didx0, semi0), (sidx1, didx1, semi1),
             (sidx2, didx2, semi2), (sidx3, didx3, semi3)]

    def istart(i, slot):
        sidx, didx, semi = slot
        pltpu.async_copy(src_hbm.at[pl.ds(base + i * CH, CH)], sidx, semi)
        pltpu.async_copy(dst_hbm.at[pl.ds(base + i * CH, CH)], didx, semi)

    def iwait(i, slot):
        sidx, didx, semi = slot
        pltpu.make_async_copy(
            src_hbm.at[pl.ds(base + i * CH, CH)], sidx, semi).wait()
        pltpu.make_async_copy(
            dst_hbm.at[pl.ds(base + i * CH, CH)], didx, semi).wait()

    def gather(fel_b, er_b, sem, slot):
        sidx, didx, _ = slot
        pltpu.async_copy(fel_hbm.at[sidx], fel_b, sem)
        pltpu.async_copy(er_hbm.at[didx], er_b, sem)

    def gwait(fel_b, er_b, sem, slot):
        sidx, didx, _ = slot
        pltpu.make_async_copy(fel_hbm.at[sidx], fel_b, sem).wait()
        pltpu.make_async_copy(er_hbm.at[didx], er_b, sem).wait()

    def compute_scatter(fel_t, er_t, didx_b):
        @plsc.parallel_loop(0, CH, unroll=4)
        def _edge_loop(e):
            for k in range(4):
                f = fel_t[e, pl.ds(k * 16, 16)]
                a = fel_t[e, pl.ds(64 + k * 16, 16)]
                b = er_t[e, pl.ds(k * 16, 16)]
                t = a + b
                t = jnp.maximum(t, 0.2 * t)
                ex = jnp.exp(t)
                fel_t[e, pl.ds(64 + k * 16, 16)] = ex
                fel_t[e, pl.ds(k * 16, 16)] = ex * f
        pltpu.sync_copy(fel_t, acc_sh.at[didx_b], add=True)

    def process(i, m):
        fel_b, er_b, sem = rows[m % 2]
        gwait(fel_b, er_b, sem, slots[m % 4])
        compute_scatter(fel_b, er_b, slots[m % 4][1])
        if isinstance(i, int):
            if i + 4 < NCHUNK:
                istart(i + 4, slots[m % 4])
            if i + 2 < NCHUNK:
                iwait(i + 2, slots[(m + 2) % 4])
                gather(fel_b, er_b, sem, slots[(m + 2) % 4])
        else:
            istart(i + 4, slots[m % 4])
            iwait(i + 2, slots[(m + 2) % 4])
            gather(fel_b, er_b, sem, slots[(m + 2) % 4])

    for i in range(4):
        istart(i, slots[i])
    iwait(0, slots[0])
    gather(fel0, er0, sem0, slots[0])
    iwait(1, slots[1])
    gather(fel1, er1, sem1, slots[1])

    def quad(jj, carry):
        i0 = 4 * jj
        for m in range(4):
            process(i0 + m, m)
        return carry

    lax.fori_loop(0, NCHUNK // 4 - 1, quad, 0)
    for i in range(NCHUNK // 4 * 4 - 4, NCHUNK):
        process(i, i % 4)

    plsc.subcore_barrier()
    pltpu.sync_copy(acc_sh.at[pl.ds(s * RPT, RPT)],
                    acc_out.at[c, pl.ds(s * RPT, RPT)])


@functools.cache
def _sc_edge_kernel():
  return pl.kernel(
    _sc_edge_body,
    out_type=jax.ShapeDtypeStruct((NC, NP, 128), jnp.float32),
    mesh=plsc.VectorSubcoreMesh(core_axis_name="c", subcore_axis_name="s",
                                num_cores=NC, num_subcores=NS),
    scratch_types=[
        pltpu.VMEM_SHARED((NP, 128), jnp.float32),
        pltpu.VMEM((CH, 128), jnp.float32),
        pltpu.VMEM((CH, 128), jnp.float32),
        pltpu.VMEM((CH, 128), jnp.float32),
        pltpu.VMEM((CH, 128), jnp.float32),
        pltpu.VMEM((CH,), jnp.int32),
        pltpu.VMEM((CH,), jnp.int32),
        pltpu.VMEM((CH,), jnp.int32),
        pltpu.VMEM((CH,), jnp.int32),
        pltpu.VMEM((CH,), jnp.int32),
        pltpu.VMEM((CH,), jnp.int32),
        pltpu.VMEM((CH,), jnp.int32),
        pltpu.VMEM((CH,), jnp.int32),
        pltpu.SemaphoreType.DMA,
        pltpu.SemaphoreType.DMA,
        pltpu.SemaphoreType.DMA,
        pltpu.SemaphoreType.DMA,
        pltpu.SemaphoreType.DMA,
        pltpu.SemaphoreType.DMA,
    ],
  )


def kernel(x, edge_index, init_weight_y, W0, al0, ar0, W1, attn_l, attn_r,
           tau1, tau2, W_lin):
    src = edge_index[0]
    dst = edge_index[1]

    bcast = jnp.kron(jnp.eye(HH, dtype=jnp.float32),
                     jnp.ones((FF, FF), dtype=jnp.float32))
    Al = al0.reshape(HH * FF, 1) * bcast
    Ar = ar0.reshape(HH * FF, 1) * bcast
    A1l = attn_l.reshape(HH * FF, 1) * bcast
    A1r = attn_r.reshape(HH * FF, 1) * bcast
    W1bd = jnp.kron(jnp.eye(HH, dtype=jnp.float32), W1)
    taus = jnp.stack([tau1, tau2]).reshape(1, 2).astype(jnp.float32)
    zrow = jnp.zeros((RPT, 128), jnp.float32)

    sc_edge = _sc_edge_kernel()
    fel, er64 = _tc_prep(x, W0, Al, Ar)
    acc0 = sc_edge(fel, er64, src, dst, zrow)
    fel1, er1, z8 = _tc_mid(acc0, W1bd, A1l, A1r, W_lin, taus)
    acc1 = sc_edge(fel1, er1, src, dst, zrow)
    out = _tc_out(acc1, fel1, z8, W_lin)
    return out
